# baseline (speedup 1.0000x reference)
import functools

import jax
import jax.numpy as jnp
from jax.experimental import pallas as pl
from jax.experimental.pallas import tpu as pltpu
from jax.experimental.pallas import tpu_sc as plsc

N_CORES = 2
N_SUBCORES = 16
NW = N_CORES * N_SUBCORES
BATCH = 16384
D_MODEL = 128
N_REL = 1000
REL_PAD = 1024
CHUNK = 128
B_PER_W = BATCH // NW
N_CHUNKS = B_PER_W // CHUNK
TC_BLOCK = 128


def _make_sc_kernel():
  mesh = plsc.VectorSubcoreMesh(
      core_axis_name="c", subcore_axis_name="s",
      num_cores=N_CORES, num_subcores=N_SUBCORES)
  out_type = (jax.ShapeDtypeStruct((BATCH, D_MODEL), jnp.float32),) * 2

  @functools.partial(
      pl.kernel,
      out_type=out_type,
      mesh=mesh,
      scratch_types=[
          pltpu.VMEM((N_CHUNKS, CHUNK), jnp.int32),
          pltpu.VMEM((N_CHUNKS, CHUNK), jnp.int32),
          pltpu.VMEM((CHUNK, D_MODEL), jnp.float32),
          pltpu.VMEM((CHUNK, D_MODEL), jnp.float32),
          pltpu.VMEM((CHUNK, D_MODEL), jnp.float32),
          pltpu.VMEM((CHUNK, D_MODEL), jnp.float32),
          pltpu.SemaphoreType.DMA,
          pltpu.SemaphoreType.DMA,
          pltpu.SemaphoreType.DMA,
          pltpu.SemaphoreType.DMA,
          pltpu.SemaphoreType.DMA,
          pltpu.SemaphoreType.DMA,
          pltpu.SemaphoreType.DMA,
          pltpu.SemaphoreType.DMA,
      ],
  )
  def ent_gather(qe_h, oe_h, ent_h,
                 out_qe, out_oe,
                 idx_q, idx_o,
                 buf0, buf1, buf2, buf3,
                 gs0, gs1, gs2, gs3, os0, os1, os2, os3):
    wid = jax.lax.axis_index("s") * N_CORES + jax.lax.axis_index("c")
    idx_base = wid * N_CHUNKS
    row_base = wid * B_PER_W

    pltpu.sync_copy(qe_h.at[pl.ds(idx_base, N_CHUNKS)], idx_q)
    pltpu.sync_copy(oe_h.at[pl.ds(idx_base, N_CHUNKS)], idx_o)

    tasks = []
    for idx_ref, out in ((idx_q, out_qe), (idx_o, out_oe)):
      for c in range(N_CHUNKS):
        tasks.append((idx_ref.at[c], out.at[pl.ds(row_base + c * CHUNK, CHUNK)]))

    bufs = (buf0, buf1, buf2, buf3)
    gsems = (gs0, gs1, gs2, gs3)
    osems = (os0, os1, os2, os3)
    nbuf = len(bufs)
    nt = len(tasks)

    def start_gather(t):
      return pltpu.async_copy(ent_h.at[tasks[t][0]], bufs[t % nbuf],
                              gsems[t % nbuf])

    g = {t: start_gather(t) for t in range(min(3, nt))}
    o = {}
    for t in range(nt):
      g[t].wait()
      o[t] = pltpu.async_copy(bufs[t % nbuf], tasks[t][1], osems[t % nbuf])
      if t + 3 < nt:
        if t >= 1:
          o[t - 1].wait()
        g[t + 3] = start_gather(t + 3)
    for t in range(max(0, nt - 4), nt):
      o[t].wait()

  return ent_gather


def _rel_tc_body(idx_ref, rel_ref, out_ref):
  row = idx_ref[0]
  rel_ids = jax.lax.broadcasted_iota(jnp.int32, (REL_PAD, TC_BLOCK), 0)
  onehot_t = (rel_ids == row).astype(jnp.bfloat16)
  out_ref[...] = jax.lax.dot_general(
      onehot_t, rel_ref[...], (((0,), (0,)), ((), ())),
      preferred_element_type=jnp.float32)


_SC_KERNEL = _make_sc_kernel()

_REL_TC = pl.pallas_call(
    _rel_tc_body,
    grid=(BATCH // TC_BLOCK,),
    in_specs=[
        pl.BlockSpec((1, 1, TC_BLOCK), lambda i: (i, 0, 0)),
        pl.BlockSpec((REL_PAD, D_MODEL), lambda i: (0, 0)),
    ],
    out_specs=pl.BlockSpec((TC_BLOCK, D_MODEL), lambda i: (i, 0)),
    out_shape=jax.ShapeDtypeStruct((BATCH, D_MODEL), jnp.float32),
)


def kernel(query_entities, query_relations, obj_entities, ent_table, rel_table):
  qe = query_entities.reshape(NW * N_CHUNKS, CHUNK)
  oe = obj_entities.reshape(NW * N_CHUNKS, CHUNK)
  out_qe, out_oe = _SC_KERNEL(qe, oe, ent_table)

  rel_pad = jnp.zeros((REL_PAD, D_MODEL), jnp.bfloat16)
  rel_pad = rel_pad.at[:N_REL].set(rel_table.astype(jnp.bfloat16))
  out_qr = _REL_TC(query_relations.reshape(BATCH // TC_BLOCK, 1, TC_BLOCK),
                   rel_pad)
  return (out_qe, out_qr, out_oe)


# speedup vs baseline: 2.0387x; 2.0387x over previous
import functools

import jax
import jax.numpy as jnp
from jax.experimental import pallas as pl
from jax.experimental.pallas import tpu as pltpu
from jax.experimental.pallas import tpu_sc as plsc

N_CORES = 2
N_SUBCORES = 16
NW = N_CORES * N_SUBCORES
BATCH = 16384
D_MODEL = 128
CHUNK = 128
B_PER_W = BATCH // NW
N_CHUNKS = B_PER_W // CHUNK
NBUF = 6


def _make_kernel():
  mesh = plsc.VectorSubcoreMesh(
      core_axis_name="c", subcore_axis_name="s",
      num_cores=N_CORES, num_subcores=N_SUBCORES)
  out_type = (jax.ShapeDtypeStruct((BATCH, D_MODEL), jnp.float32),) * 3
  scratch = (
      [pltpu.VMEM((N_CHUNKS, CHUNK), jnp.int32)] * 3
      + [pltpu.VMEM((CHUNK, D_MODEL), jnp.float32)] * NBUF
      + [pltpu.SemaphoreType.DMA] * (2 * NBUF + 3)
  )

  @functools.partial(
      pl.kernel, out_type=out_type, mesh=mesh, scratch_types=scratch)
  def trans_e_gather(qe_h, qr_h, oe_h, ent_h, rel_h,
                     out_qe, out_qr, out_oe, *scr):
    idx_q, idx_r, idx_o = scr[0:3]
    bufs = scr[3:3 + NBUF]
    gsems = scr[3 + NBUF:3 + 2 * NBUF]
    osems = scr[3 + 2 * NBUF:3 + 3 * NBUF]
    isems = scr[3 + 3 * NBUF:]

    wid = jax.lax.axis_index("s") * N_CORES + jax.lax.axis_index("c")
    idx_base = wid * N_CHUNKS
    row_base = wid * B_PER_W

    icopies = [
        pltpu.async_copy(src.at[pl.ds(idx_base, N_CHUNKS)], dst, sem)
        for src, dst, sem in ((qe_h, idx_q, isems[0]),
                              (qr_h, idx_r, isems[1]),
                              (oe_h, idx_o, isems[2]))
    ]

    tasks = []
    for idx_ref, tab, out in ((idx_q, ent_h, out_qe),
                              (idx_r, rel_h, out_qr),
                              (idx_o, ent_h, out_oe)):
      for c in range(N_CHUNKS):
        tasks.append((idx_ref.at[c], tab, out.at[pl.ds(row_base + c * CHUNK, CHUNK)]))
    nt = len(tasks)

    def start_gather(t):
      idx_s, tab, _ = tasks[t]
      return pltpu.async_copy(tab.at[idx_s], bufs[t % NBUF], gsems[t % NBUF])

    g = {}
    o = {}
    for t in range(4):
      if t == 0:
        icopies[0].wait()
      g[t] = start_gather(t)
    icopies[1].wait()
    icopies[2].wait()
    for t in range(nt):
      g[t].wait()
      o[t] = pltpu.async_copy(bufs[t % NBUF], tasks[t][2], osems[t % NBUF])
      if t + 4 < nt:
        if t >= 2:
          o[t - 2].wait()
        g[t + 4] = start_gather(t + 4)
    for t in range(nt - 6, nt):
      o[t].wait()

  return trans_e_gather


_KERNEL = _make_kernel()


def kernel(query_entities, query_relations, obj_entities, ent_table, rel_table):
  qe = query_entities.reshape(NW * N_CHUNKS, CHUNK)
  qr = query_relations.reshape(NW * N_CHUNKS, CHUNK)
  oe = obj_entities.reshape(NW * N_CHUNKS, CHUNK)
  return _KERNEL(qe, qr, oe, ent_table, rel_table)
